# Initial kernel scaffold; baseline (speedup 1.0000x reference)
import jax
import jax.numpy as jnp
from jax.experimental import pallas as pl


def kernel(in_nodes_features, edge_index, W_proj, att_W1, att_b1, att_W2, att_b2, edge_W1, edge_b1, edge_W2, edge_b2, theta, bias, W_skip):
    raise NotImplementedError("write your pallas kernel here")


# speedup vs baseline: 2.5760x; 2.5760x over previous
import functools

import jax
import jax.numpy as jnp
from jax import lax
from jax.experimental import pallas as pl
from jax.experimental.pallas import tpu as pltpu
from jax.experimental.pallas import tpu_sc as plsc

_N = 10000
_E = 320000
_F_IN = 128
_F_OUT = 32
_HID = 64

_NC = 2
_NS = 16
_NW = _NC * _NS
_CH = 128
_NCHUNKS = _E // _CH
_CPW = 80
_NCH_PAD = _NW * _CPW
_NP = 10112
_ROWS_PT = _NP // _NS

_TB = 1280
_NB = _E // _TB

@functools.lru_cache(maxsize=None)
def _sc_mesh():
    return plsc.VectorSubcoreMesh(
        core_axis_name="c", subcore_axis_name="s",
        num_cores=_NC, num_subcores=_NS,
    )


def _proj_body(x_ref, wp_ref, ws_ref, projw_ref, proj_ref, skip_ref):
    x = x_ref[...]
    p = jnp.dot(x, wp_ref[...], preferred_element_type=jnp.float32)
    proj_ref[...] = p
    skip_ref[...] = jnp.dot(x, ws_ref[...], preferred_element_type=jnp.float32)
    projw_ref[...] = jnp.concatenate(
        [p, jnp.zeros((_N, 128 - _F_OUT), jnp.float32)], axis=1)


_proj_call = pl.pallas_call(
    _proj_body,
    out_shape=[
        jax.ShapeDtypeStruct((_N, 128), jnp.float32),
        jax.ShapeDtypeStruct((_N, _F_OUT), jnp.float32),
        jax.ShapeDtypeStruct((_N, _F_OUT), jnp.float32),
    ],
)


def _sc_gather_body(proj, i0, i1, i2, hi, hj, hk,
                    x0, x1, x2, r0, r1, r2, s0, s1, s2):
    wid = lax.axis_index("s") * _NC + lax.axis_index("c")
    start = wid * _CPW
    pltpu.sync_copy(i0.at[pl.ds(start, _CPW), :], x0)
    pltpu.sync_copy(i1.at[pl.ds(start, _CPW), :], x1)
    pltpu.sync_copy(i2.at[pl.ds(start, _CPW), :], x2)

    def step(j, carry):
        ch = start + j

        @pl.when(ch < _NCHUNKS)
        def _():
            c0 = pltpu.async_copy(proj.at[x0.at[j]], r0, s0)
            c1 = pltpu.async_copy(proj.at[x1.at[j]], r1, s1)
            c2 = pltpu.async_copy(proj.at[x2.at[j]], r2, s2)
            c0.wait()
            c1.wait()
            c2.wait()
            base = ch * _CH
            pltpu.sync_copy(r0, hi.at[pl.ds(base, _CH), :])
            pltpu.sync_copy(r1, hj.at[pl.ds(base, _CH), :])
            pltpu.sync_copy(r2, hk.at[pl.ds(base, _CH), :])

        return carry

    lax.fori_loop(0, _CPW, step, 0)


@functools.lru_cache(maxsize=None)
def _sc_gather():
    return pl.kernel(
        _sc_gather_body,
        out_type=[jax.ShapeDtypeStruct((_E, 128), jnp.float32)] * 3,
        mesh=_sc_mesh(),
        scratch_types=[
            pltpu.VMEM((_CPW, _CH), jnp.int32),
            pltpu.VMEM((_CPW, _CH), jnp.int32),
            pltpu.VMEM((_CPW, _CH), jnp.int32),
            pltpu.VMEM((_CH, 128), jnp.float32),
            pltpu.VMEM((_CH, 128), jnp.float32),
            pltpu.VMEM((_CH, 128), jnp.float32),
            pltpu.SemaphoreType.DMA,
            pltpu.SemaphoreType.DMA,
            pltpu.SemaphoreType.DMA,
        ],
    )


def _scores_body(hi_ref, hj_ref, hk_ref, w1a, w1b, w1c, b1, w2r, b2,
                 s_ref, m_ref):
    i = pl.program_id(0)
    hi = hi_ref[...][:, :_F_OUT]
    hj = hj_ref[...][:, :_F_OUT]
    hk = hk_ref[...][:, :_F_OUT]
    h = jnp.dot(hi, w1a[...], preferred_element_type=jnp.float32)
    h = h + jnp.dot(hj, w1b[...], preferred_element_type=jnp.float32)
    h = h + jnp.dot(hk, w1c[...], preferred_element_type=jnp.float32)
    h = jnp.maximum(h + b1[...], 0.0)
    s = jnp.sum(h * w2r[...], axis=1, keepdims=True) + b2[...]
    s = jnp.where(s > 0, s, 0.2 * s)
    s_ref[...] = s
    bm = jnp.max(s).reshape(1, 1)
    m_ref[...] = jnp.where(i == 0, bm, jnp.maximum(m_ref[...], bm))


_scores_call = pl.pallas_call(
    _scores_body,
    grid=(_NB,),
    in_specs=[
        pl.BlockSpec((_TB, 128), lambda i: (i, 0)),
        pl.BlockSpec((_TB, 128), lambda i: (i, 0)),
        pl.BlockSpec((_TB, 128), lambda i: (i, 0)),
        pl.BlockSpec((_F_OUT, _HID), lambda i: (0, 0)),
        pl.BlockSpec((_F_OUT, _HID), lambda i: (0, 0)),
        pl.BlockSpec((_F_OUT, _HID), lambda i: (0, 0)),
        pl.BlockSpec((1, _HID), lambda i: (0, 0)),
        pl.BlockSpec((1, _HID), lambda i: (0, 0)),
        pl.BlockSpec((1, 1), lambda i: (0, 0)),
    ],
    out_specs=[
        pl.BlockSpec((_TB, 1), lambda i: (i, 0)),
        pl.BlockSpec((1, 1), lambda i: (0, 0)),
    ],
    out_shape=[
        jax.ShapeDtypeStruct((_E, 1), jnp.float32),
        jax.ShapeDtypeStruct((1, 1), jnp.float32),
    ],
)


def _weight_body(hj_ref, hk_ref, s_ref, m_ref, e1a, e1b, eb1, e2, eb2,
                 wn_ref, z_ref):
    i = pl.program_id(0)
    hj = hj_ref[...][:, :_F_OUT]
    hk = hk_ref[...][:, :_F_OUT]
    e1 = jnp.dot(hj, e1a[...], preferred_element_type=jnp.float32)
    e1 = e1 + jnp.dot(hk, e1b[...], preferred_element_type=jnp.float32)
    e1 = e1 + eb1[...]
    e1 = jnp.where(e1 > 0, e1, 0.2 * e1)
    npd = jnp.dot(e1, e2[...], preferred_element_type=jnp.float32) + eb2[...]
    w = jnp.exp(s_ref[...] - m_ref[...])
    wn_ref[...] = jnp.concatenate(
        [npd * w, jnp.zeros((_TB, 128 - _F_OUT), jnp.float32)], axis=1)
    bz = jnp.sum(w).reshape(1, 1)
    z_ref[...] = jnp.where(i == 0, bz, z_ref[...] + bz)


_weight_call = pl.pallas_call(
    _weight_body,
    grid=(_NB,),
    in_specs=[
        pl.BlockSpec((_TB, 128), lambda i: (i, 0)),
        pl.BlockSpec((_TB, 128), lambda i: (i, 0)),
        pl.BlockSpec((_TB, 1), lambda i: (i, 0)),
        pl.BlockSpec((1, 1), lambda i: (0, 0)),
        pl.BlockSpec((_F_OUT, _F_OUT), lambda i: (0, 0)),
        pl.BlockSpec((_F_OUT, _F_OUT), lambda i: (0, 0)),
        pl.BlockSpec((1, _F_OUT), lambda i: (0, 0)),
        pl.BlockSpec((_F_OUT, _F_OUT), lambda i: (0, 0)),
        pl.BlockSpec((1, _F_OUT), lambda i: (0, 0)),
    ],
    out_specs=[
        pl.BlockSpec((_TB, 128), lambda i: (i, 0)),
        pl.BlockSpec((1, 1), lambda i: (0, 0)),
    ],
    out_shape=[
        jax.ShapeDtypeStruct((_E, 128), jnp.float32),
        jax.ShapeDtypeStruct((1, 1), jnp.float32),
    ],
)


def _sc_scatter_body(wn, si, out, xi, rows, acc, sem):
    cid = lax.axis_index("c")
    sid = lax.axis_index("s")
    wid = sid * _NC + cid
    start = wid * _CPW

    z16 = jnp.zeros((16,), jnp.float32)

    def zstep(r, carry):
        for k in range(8):
            rows[r, pl.ds(16 * k, 16)] = z16
        return carry

    lax.fori_loop(0, _CH, zstep, 0)
    abase = sid * _ROWS_PT
    for k in range(4):
        pltpu.sync_copy(rows, acc.at[pl.ds(abase + 128 * k, 128), :])
    pltpu.sync_copy(rows.at[pl.ds(0, 120), :],
                    acc.at[pl.ds(abase + 512, 120), :])
    plsc.subcore_barrier()

    pltpu.sync_copy(si.at[pl.ds(start, _CPW), :], xi)

    def step(j, carry):
        ch = start + j

        @pl.when(ch < _NCHUNKS)
        def _():
            pltpu.sync_copy(wn.at[pl.ds(ch * _CH, _CH), :], rows)
            pltpu.sync_copy(rows, acc.at[xi.at[j]], add=True)

        return carry

    lax.fori_loop(0, _CPW, step, 0)
    plsc.subcore_barrier()
    pltpu.sync_copy(acc.at[pl.ds(sid * _ROWS_PT, _ROWS_PT), :],
                    out.at[cid, pl.ds(sid * _ROWS_PT, _ROWS_PT), :])


@functools.lru_cache(maxsize=None)
def _sc_scatter():
    return pl.kernel(
        _sc_scatter_body,
        out_type=jax.ShapeDtypeStruct((_NC, _NP, 128), jnp.float32),
        mesh=_sc_mesh(),
        scratch_types=[
            pltpu.VMEM((_CPW, _CH), jnp.int32),
            pltpu.VMEM((_CH, 128), jnp.float32),
            pltpu.VMEM_SHARED((_NP, 128), jnp.float32),
            pltpu.SemaphoreType.DMA,
        ],
    )


def _final_body(proj_ref, skip_ref, part_ref, z_ref, theta_ref, bias_ref,
                out_ref):
    parts = part_ref[...]
    wsum = (parts[0, :_N, :_F_OUT] + parts[1, :_N, :_F_OUT]) * (1.0 / z_ref[...])
    t = proj_ref[...] * theta_ref[...] + wsum + skip_ref[...] + bias_ref[...]
    out_ref[...] = jnp.where(t > 0, t, jnp.exp(jnp.minimum(t, 0.0)) - 1.0)


_final_call = pl.pallas_call(
    _final_body,
    out_shape=jax.ShapeDtypeStruct((_N, _F_OUT), jnp.float32),
)


def kernel(in_nodes_features, edge_index, W_proj, att_W1, att_b1, att_W2,
           att_b2, edge_W1, edge_b1, edge_W2, edge_b2, theta, bias, W_skip):
    x = in_nodes_features
    projw, proj, skip = _proj_call(x, W_proj, W_skip)

    pad = _NCH_PAD * _CH - _E
    idxp = jnp.pad(edge_index, ((0, 0), (0, pad)))
    i0 = idxp[0].reshape(_NCH_PAD, _CH)
    i1 = idxp[1].reshape(_NCH_PAD, _CH)
    i2 = idxp[2].reshape(_NCH_PAD, _CH)

    hi, hj, hk = _sc_gather()(projw, i0, i1, i2)

    s, m = _scores_call(
        hi, hj, hk,
        att_W1[0:_F_OUT], att_W1[_F_OUT:2 * _F_OUT], att_W1[2 * _F_OUT:],
        att_b1.reshape(1, _HID), att_W2.reshape(1, _HID),
        att_b2.reshape(1, 1),
    )

    wn, z = _weight_call(
        hj, hk, s, m,
        edge_W1[0:_F_OUT], edge_W1[_F_OUT:],
        edge_b1.reshape(1, _F_OUT), edge_W2, edge_b2.reshape(1, _F_OUT),
    )

    partials = _sc_scatter()(wn, i0)

    out = _final_call(proj, skip, partials, z, theta.reshape(1, _F_OUT),
                      bias.reshape(1, _F_OUT))
    return out
